# baseline (speedup 1.0000x reference)
import jax
import jax.numpy as jnp
from jax import lax
from jax.experimental import pallas as pl
from jax.experimental.pallas import tpu as pltpu
from jax.experimental.pallas import tpu_sc as plsc

T = 26
VOCAB = 100000
D = 32
B = 1024
L = 20
PW = 128

NC, NS = 2, 16
NW = NC * NS
NBAGS = T * B
BPW = NBAGS // NW
CB = 32
NCHUNK = BPW // CB
IPC = CB * L
GCHUNK = 128
NG = IPC // GCHUNK
INV_L = 1.0 / L


def _sc_body(table, idx_hbm, out_hbm, idx_v, rows_v, out_v, sem):
    i32 = jnp.int32
    wid = lax.axis_index("s") * i32(NC) + lax.axis_index("c")
    bag0 = wid * i32(BPW)

    def chunk_body(c, carry):
        base_bag = bag0 + c * i32(CB)
        base_idx = base_bag * i32(L)
        pltpu.sync_copy(idx_hbm.at[pl.ds(base_idx, IPC)], idx_v)
        descs = []
        for j in range(NG):
            descs.append(pltpu.async_copy(
                table.at[idx_v.at[pl.ds(j * GCHUNK, GCHUNK)]],
                rows_v.at[pl.ds(j * GCHUNK, GCHUNK)],
                sem))
        for d in descs:
            d.wait()

        def bag_body(b, carry2):
            r0 = b * i32(L)
            acc0 = rows_v[r0, pl.ds(0, 16)]
            acc1 = rows_v[r0, pl.ds(16, 16)]
            for l in range(1, L):
                acc0 = acc0 + rows_v[r0 + i32(l), pl.ds(0, 16)]
                acc1 = acc1 + rows_v[r0 + i32(l), pl.ds(16, 16)]
            out_v[b, pl.ds(0, 16)] = acc0 * INV_L
            out_v[b, pl.ds(16, 16)] = acc1 * INV_L
            return carry2

        lax.fori_loop(i32(0), i32(CB), bag_body, i32(0))
        pltpu.sync_copy(out_v, out_hbm.at[pl.ds(base_bag, CB)])
        return carry

    lax.fori_loop(i32(0), i32(NCHUNK), chunk_body, i32(0))


def kernel(weight, indices, offsets):
    del offsets
    wpad = jnp.pad(weight.reshape(T * VOCAB, D), ((0, 0), (0, PW - D)))
    idx = indices.astype(jnp.int32)
    mesh = plsc.VectorSubcoreMesh(core_axis_name="c", subcore_axis_name="s")
    run = pl.kernel(
        _sc_body,
        out_type=jax.ShapeDtypeStruct((NBAGS, D), jnp.float32),
        mesh=mesh,
        scratch_types=[
            pltpu.VMEM((IPC,), jnp.int32),
            pltpu.VMEM((IPC, PW), jnp.float32),
            pltpu.VMEM((CB, D), jnp.float32),
            pltpu.SemaphoreType.DMA,
        ],
        compiler_params=pltpu.CompilerParams(
            use_tc_tiling_on_sc=True, needs_layout_passes=False),
    )
    out_flat = run(wpad, idx)
    return out_flat.reshape(T, B, D).transpose(1, 0, 2).reshape(B, T * D)
